# baseline (speedup 1.0000x reference)
import functools

import jax
import jax.numpy as jnp
from jax import lax
from jax.experimental import pallas as pl
from jax.experimental.pallas import tpu as pltpu
from jax.experimental.pallas import tpu_sc as plsc

VOCAB = 1000000
EMB = 16
BATCH = 16384

NC = 2
NS = 16
NW = NC * NS
BPW = BATCH // NW
NCHUNK = 128
NJ = BPW // NCHUNK


def _rsqrt_nr(x):
    i = lax.bitcast_convert_type(x, jnp.int32)
    i = jnp.int32(0x5F3759DF) - lax.shift_right_logical(i, 1)
    y = lax.bitcast_convert_type(i, jnp.float32)
    half = jnp.float32(0.5) * x
    for _ in range(3):
        y = y * (jnp.float32(1.5) - half * y * y)
    return y


def _make_sc_kernel():
    mesh = plsc.VectorSubcoreMesh(core_axis_name="c", subcore_axis_name="s")

    @functools.partial(
        pl.kernel,
        mesh=mesh,
        out_type=jax.ShapeDtypeStruct((BATCH,), jnp.float32),
        compiler_params=pltpu.CompilerParams(
            needs_layout_passes=False, use_tc_tiling_on_sc=False),
        scratch_types=[
            pltpu.VMEM((NJ, NCHUNK), jnp.int32),
            pltpu.VMEM((NJ, NCHUNK), jnp.int32),
            pltpu.VMEM((BPW, EMB), jnp.float32),
            pltpu.VMEM((BPW, EMB), jnp.float32),
            pltpu.VMEM((BPW,), jnp.float32),
            pltpu.SemaphoreType.DMA,
        ],
    )
    def sc_kernel(inp_hbm, table_hbm, out_hbm, ia, ib, ar, br, outv, sem):
        wid = lax.axis_index("s") * NC + lax.axis_index("c")
        base = wid * BPW

        pltpu.sync_copy(inp_hbm.at[0, wid], ia)
        pltpu.sync_copy(inp_hbm.at[1, wid], ib)

        copies = []
        for j in range(NJ):
            copies.append(pltpu.async_copy(
                table_hbm.at[ia.at[j]], ar.at[pl.ds(j * NCHUNK, NCHUNK)], sem))
            copies.append(pltpu.async_copy(
                table_hbm.at[ib.at[j]], br.at[pl.ds(j * NCHUNK, NCHUNK)], sem))
        for cp in copies:
            cp.wait()

        lane = lax.iota(jnp.int32, 16)
        eps2 = jnp.full((16,), 1e-16, jnp.float32)

        def body(c, _):
            rows = c * 16 + lane
            dot = jnp.zeros((16,), jnp.float32)
            a2 = jnp.zeros((16,), jnp.float32)
            b2 = jnp.zeros((16,), jnp.float32)
            for d in range(EMB):
                cols = jnp.full((16,), d, jnp.int32)
                av = plsc.load_gather(ar, [rows, cols])
                bv = plsc.load_gather(br, [rows, cols])
                dot = dot + av * bv
                a2 = a2 + av * av
                b2 = b2 + bv * bv
            denom2 = jnp.maximum(a2 * b2, eps2)
            outv[pl.ds(c * 16, 16)] = dot * _rsqrt_nr(denom2)
            return 0

        lax.fori_loop(0, BPW // 16, body, 0)

        pltpu.sync_copy(outv, out_hbm.at[pl.ds(base, BPW)])

    return sc_kernel


_sc_kernel = _make_sc_kernel()


def kernel(input, table):
    inp = input.T.reshape(2, NW, NJ, NCHUNK)
    return _sc_kernel(inp, table)
